# Initial kernel scaffold; baseline (speedup 1.0000x reference)
import jax
import jax.numpy as jnp
from jax.experimental import pallas as pl


def kernel(tokens, lut):
    raise NotImplementedError("write your pallas kernel here")


# speedup vs baseline: 1.7793x; 1.7793x over previous
import functools

import jax
import jax.numpy as jnp
from jax import lax
from jax.experimental import pallas as pl
from jax.experimental.pallas import tpu as pltpu
from jax.experimental.pallas import tpu_sc as plsc

_HIDDEN = 2048
_TOTAL = 16384
_NW = 32
_B_PER_W = _TOTAL // _NW
_CHUNK = 16
_NBUF = 2
_N_CHUNKS = _B_PER_W // _CHUNK

_mesh = plsc.VectorSubcoreMesh(core_axis_name="c", subcore_axis_name="s")


@functools.partial(
    pl.kernel,
    mesh=_mesh,
    out_type=jax.ShapeDtypeStruct((_TOTAL, _HIDDEN), jnp.float32),
    scratch_types=[
        pltpu.VMEM((_B_PER_W,), jnp.int32),
        pltpu.VMEM((_NBUF, _CHUNK, _HIDDEN), jnp.float32),
        pltpu.SemaphoreType.DMA,
        pltpu.SemaphoreType.DMA,
        pltpu.SemaphoreType.DMA,
        pltpu.SemaphoreType.DMA,
    ],
)
def _emb_lookup(tokens_hbm, lut_hbm, out_hbm, idx_v, rows_v, g0, g1, w0, w1):
    wid = lax.axis_index("s") * 2 + lax.axis_index("c")
    base = wid * _B_PER_W
    pltpu.sync_copy(tokens_hbm.at[pl.ds(base, _B_PER_W)], idx_v)

    gsems = [g0, g1]
    wsems = [w0, w1]

    def gather(c, b):
        return pltpu.make_async_copy(
            lut_hbm.at[idx_v.at[pl.ds(c * _CHUNK, _CHUNK)]],
            rows_v.at[b],
            gsems[b],
        )

    def writeback(c, b):
        return pltpu.make_async_copy(
            rows_v.at[b],
            out_hbm.at[pl.ds(base + c * _CHUNK, _CHUNK)],
            wsems[b],
        )

    for b in range(_NBUF):
        gather(b, b).start()

    def body(i, _):
        for b in range(_NBUF):
            c = i * _NBUF + b
            gather(c, b).wait()
            writeback(c, b).start()
            nc = c + _NBUF
            writeback(c, b).wait()

            @pl.when(nc < _N_CHUNKS)
            def _():
                gather(nc, b).start()

        return 0

    lax.fori_loop(0, _N_CHUNKS // _NBUF, body, 0)


def kernel(tokens, lut):
    flat = tokens.reshape(-1).astype(jnp.int32)
    out = _emb_lookup(flat, lut)
    return out.reshape(tokens.shape + (_HIDDEN,))
